# Initial kernel scaffold; baseline (speedup 1.0000x reference)
import functools

import jax
import jax.numpy as jnp
from jax import lax
from jax.experimental import pallas as pl
from jax.experimental.pallas import tpu as pltpu

_ND = 512
_N = 256
_NL = 5
_E = _N * (_N - 1)
_SEG = _N - 1
_BLK = 8 * _SEG
_NB = _E // _BLK
_SPB = _BLK // _SEG

_F32 = jnp.float32


def _dot(a, b):
    return jnp.dot(a, b, preferred_element_type=_F32)


def _ln(x, g, b):
    m = jnp.mean(x, axis=-1, keepdims=True)
    v = jnp.mean((x - m) ** 2, axis=-1, keepdims=True)
    return (x - m) * lax.rsqrt(v + 1e-5) * g + b


def _gru_core(x, h, Wir, Whr, Wiz, Whz, Win, Whn, br, bz, bin_, bhn):
    r = jax.nn.sigmoid(_dot(x, Wir) + _dot(h, Whr) + br)
    z = jax.nn.sigmoid(_dot(x, Wiz) + _dot(h, Whz) + bz)
    n = jnp.tanh(_dot(x, Win) + bin_ + r * (_dot(h, Whn) + bhn))
    return (1.0 - z) * n + z * h


def _node_first_body(h_ref, Ws1, bs1, Ws2, bs2, Wo1, bo1, Wo2, bo2,
                     a_ref, b_ref):
    h = h_ref[...]
    a_ref[...] = _dot(jax.nn.relu(_dot(h, Ws1[...]) + bs1[...]), Ws2[...]) + bs2[...]
    b_ref[...] = _dot(jax.nn.relu(_dot(h, Wo1[...]) + bo1[...]), Wo2[...]) + bo2[...]


def _node_mid_body(mn_ref, h_ref, hs_ref,
                   Wir, Whr, Wiz, Whz, Win, Whn, br, bz, bin_, bhn,
                   Ws1, bs1, Ws2, bs2, Wo1, bo1, Wo2, bo2,
                   hnew_ref, hso_ref, a_ref, b_ref):
    hn = _gru_core(mn_ref[...], h_ref[...],
                   Wir[...], Whr[...], Wiz[...], Whz[...], Win[...], Whn[...],
                   br[...], bz[...], bin_[...], bhn[...])
    hnew_ref[...] = hn
    hso_ref[...] = hs_ref[...] + hn
    a_ref[...] = _dot(jax.nn.relu(_dot(hn, Ws1[...]) + bs1[...]), Ws2[...]) + bs2[...]
    b_ref[...] = _dot(jax.nn.relu(_dot(hn, Wo1[...]) + bo1[...]), Wo2[...]) + bo2[...]


def _node_last_body(mn_ref, h_ref, hs_ref,
                    Wir, Whr, Wiz, Whz, Win, Whn, br, bz, bin_, bhn,
                    fg, fb, hfin_ref):
    hn = _gru_core(mn_ref[...], h_ref[...],
                   Wir[...], Whr[...], Wiz[...], Whz[...], Win[...], Whn[...],
                   br[...], bz[...], bin_[...], bhn[...])
    hfin_ref[...] = _ln(hs_ref[...] + hn, fg[...], fb[...])


def _edge_a_body(he_ref, W1, b1, W2, b2, mn_ref):
    x = he_ref[...]
    w = _dot(jax.nn.relu(_dot(x, W1[...]) + b1[...]), W2[...]) + b2[...]
    r = lax.broadcasted_iota(jnp.int32, (_SPB, _BLK), 0)
    c = lax.broadcasted_iota(jnp.int32, (_SPB, _BLK), 1)
    seg = ((c >= r * _SEG) & (c < (r + 1) * _SEG)).astype(_F32)
    mn_ref[...] = _dot(seg, w)


def _edge_b_body(sidx_ref, oidx_ref, a_ref, b_ref, he_ref, hs_ref,
                 g, b, Wir, Whr, Wiz, Whz, Win, Whn, br, bz, bin_, bhn,
                 *rest, final):
    if final:
        fg, fb, hef_ref = rest
    else:
        henew_ref, hso_ref = rest
    s = sidx_ref[0]
    o = oidx_ref[0]
    lanes = lax.broadcasted_iota(jnp.int32, (_BLK, _N), 1)
    ohs = (s == lanes).astype(_F32)
    oho = (o == lanes).astype(_F32)
    cc = _dot(ohs, a_ref[...]) + _dot(oho, b_ref[...])
    mp = _ln(cc, g[...], b[...])
    he = he_ref[...]
    hn = _gru_core(mp, he,
                   Wir[...], Whr[...], Wiz[...], Whz[...], Win[...], Whn[...],
                   br[...], bz[...], bin_[...], bhn[...])
    if final:
        hef_ref[...] = _ln(hs_ref[...] + hn, fg[...], fb[...])
    else:
        henew_ref[...] = hn
        hso_ref[...] = hs_ref[...] + hn


def _full(shape):
    nd = len(shape)
    return pl.BlockSpec(shape, lambda *_: (0,) * nd)


def _node_first(h, ws):
    out = [jax.ShapeDtypeStruct((_N, _ND), _F32)] * 2
    specs = [_full((_N, _ND))] + [_full(w.shape) for w in ws]
    return pl.pallas_call(
        _node_first_body,
        out_shape=out,
        in_specs=specs,
        out_specs=[_full((_N, _ND))] * 2,
    )(h, *ws)


def _node_mid(mn, h, hs, gw, ws):
    out = [jax.ShapeDtypeStruct((_N, _ND), _F32)] * 4
    specs = [_full((_N, _ND))] * 3 + [_full(w.shape) for w in gw + ws]
    return pl.pallas_call(
        _node_mid_body,
        out_shape=out,
        in_specs=specs,
        out_specs=[_full((_N, _ND))] * 4,
    )(mn, h, hs, *gw, *ws)


def _node_last(mn, h, hs, gw, fg, fb):
    specs = ([_full((_N, _ND))] * 3 + [_full(w.shape) for w in gw]
             + [_full((1, _ND))] * 2)
    return pl.pallas_call(
        _node_last_body,
        out_shape=jax.ShapeDtypeStruct((_N, _ND), _F32),
        in_specs=specs,
        out_specs=_full((_N, _ND)),
    )(mn, h, hs, *gw, fg, fb)


def _edge_a(he, W1, b1, W2, b2):
    return pl.pallas_call(
        _edge_a_body,
        grid=(_NB,),
        out_shape=jax.ShapeDtypeStruct((_N, _ND), _F32),
        in_specs=[
            pl.BlockSpec((_BLK, _ND), lambda i: (i, 0)),
            _full((_ND, _ND)), _full((1, _ND)), _full((_ND, _ND)), _full((1, _ND)),
        ],
        out_specs=pl.BlockSpec((_SPB, _ND), lambda i: (i, 0)),
    )(he, W1, b1, W2, b2)


def _edge_b(s3, o3, a, b, he, hs, lng, lnb, gw, final, fg=None, fb=None):
    body = functools.partial(_edge_b_body, final=final)
    in_specs = [
        pl.BlockSpec((1, _BLK, 1), lambda i: (i, 0, 0)),
        pl.BlockSpec((1, _BLK, 1), lambda i: (i, 0, 0)),
        _full((_N, _ND)), _full((_N, _ND)),
        pl.BlockSpec((_BLK, _ND), lambda i: (i, 0)),
        pl.BlockSpec((_BLK, _ND), lambda i: (i, 0)),
        _full((1, _ND)), _full((1, _ND)),
    ] + [_full(w.shape) for w in gw]
    args = [s3, o3, a, b, he, hs, lng, lnb, *gw]
    if final:
        in_specs += [_full((1, _ND)), _full((1, _ND))]
        args += [fg, fb]
        out_shape = jax.ShapeDtypeStruct((_E, _ND), _F32)
        out_specs = pl.BlockSpec((_BLK, _ND), lambda i: (i, 0))
    else:
        out_shape = [jax.ShapeDtypeStruct((_E, _ND), _F32)] * 2
        out_specs = [pl.BlockSpec((_BLK, _ND), lambda i: (i, 0))] * 2
    return pl.pallas_call(
        body,
        grid=(_NB,),
        out_shape=out_shape,
        in_specs=in_specs,
        out_specs=out_specs,
    )(*args)


def _split_gru(Wih, Whh, bih, bhh):
    D = _ND
    Wir, Wiz, Win = (Wih[0:D].T, Wih[D:2 * D].T, Wih[2 * D:].T)
    Whr, Whz, Whn = (Whh[0:D].T, Whh[D:2 * D].T, Whh[2 * D:].T)
    br = (bih[0:D] + bhh[0:D]).reshape(1, D)
    bz = (bih[D:2 * D] + bhh[D:2 * D]).reshape(1, D)
    bin_ = bih[2 * D:].reshape(1, D)
    bhn = bhh[2 * D:].reshape(1, D)
    return [Wir, Whr, Wiz, Whz, Win, Whn, br, bz, bin_, bhn]


def kernel(h, h_edge, edge_index, phis_W1, phis_b1, phis_W2, phis_b2,
           phio_W1, phio_b1, phio_W2, phio_b2, phip_W1, phip_b1, phip_W2,
           phip_b2, ln_g, ln_b, gru_n_Wih, gru_n_Whh, gru_n_bih, gru_n_bhh,
           gru_e_Wih, gru_e_Whh, gru_e_bih, gru_e_bhh, final_ln_g, final_ln_b):
    s3 = edge_index[0].reshape(_NB, _BLK, 1)
    o3 = edge_index[1].reshape(_NB, _BLK, 1)
    fg = final_ln_g.reshape(1, _ND)
    fb = final_ln_b.reshape(1, _ND)

    hs = jnp.zeros((_N, _ND), _F32)
    hes = jnp.zeros((_E, _ND), _F32)
    mn = None
    he = h_edge
    hef = None

    for i in range(_NL):
        ws = [phis_W1[i], phis_b1[i].reshape(1, _ND), phis_W2[i],
              phis_b2[i].reshape(1, _ND), phio_W1[i], phio_b1[i].reshape(1, _ND),
              phio_W2[i], phio_b2[i].reshape(1, _ND)]
        gn = _split_gru(gru_n_Wih[i], gru_n_Whh[i], gru_n_bih[i], gru_n_bhh[i])
        ge = _split_gru(gru_e_Wih[i], gru_e_Whh[i], gru_e_bih[i], gru_e_bhh[i])

        if i == 0:
            a, b = _node_first(h, ws)
        else:
            h, hs, a, b = _node_mid(mn, h, hs, gn, ws)

        mn = _edge_a(he, phip_W1[i], phip_b1[i].reshape(1, _ND),
                     phip_W2[i], phip_b2[i].reshape(1, _ND))

        lng = ln_g[i].reshape(1, _ND)
        lnb = ln_b[i].reshape(1, _ND)
        if i == _NL - 1:
            hef = _edge_b(s3, o3, a, b, he, hes, lng, lnb, ge,
                          final=True, fg=fg, fb=fb)
        else:
            he, hes = _edge_b(s3, o3, a, b, he, hes, lng, lnb, ge, final=False)

    gn = _split_gru(gru_n_Wih[_NL - 1], gru_n_Whh[_NL - 1],
                    gru_n_bih[_NL - 1], gru_n_bhh[_NL - 1])
    hf = _node_last(mn, h, hs, gn, fg, fb)
    return (hf, hef)


# speedup vs baseline: 2.9797x; 2.9797x over previous
import functools

import jax
import jax.numpy as jnp
from jax import lax
from jax.experimental import pallas as pl
from jax.experimental.pallas import tpu as pltpu

_ND = 512
_N = 256
_NL = 5
_E = _N * (_N - 1)
_SEG = _N - 1
_BLK = 8 * _SEG
_NB = _E // _BLK
_SPB = _BLK // _SEG

_F32 = jnp.float32


def _dot(a, b):
    return jnp.dot(a, b, preferred_element_type=_F32)


def _ln(x, g, b):
    m = jnp.mean(x, axis=-1, keepdims=True)
    v = jnp.mean((x - m) ** 2, axis=-1, keepdims=True)
    return (x - m) * lax.rsqrt(v + 1e-5) * g + b


def _gru_core(x, h, Wir, Whr, Wiz, Whz, Win, Whn, br, bz, bin_, bhn):
    r = jax.nn.sigmoid(_dot(x, Wir) + _dot(h, Whr) + br)
    z = jax.nn.sigmoid(_dot(x, Wiz) + _dot(h, Whz) + bz)
    n = jnp.tanh(_dot(x, Win) + bin_ + r * (_dot(h, Whn) + bhn))
    return (1.0 - z) * n + z * h


def _node_first_body(h_ref, Ws1, bs1, Ws2, bs2, Wo1, bo1, Wo2, bo2,
                     a_ref, b_ref):
    h = h_ref[...]
    a_ref[...] = _dot(jax.nn.relu(_dot(h, Ws1[...]) + bs1[...]), Ws2[...]) + bs2[...]
    b_ref[...] = _dot(jax.nn.relu(_dot(h, Wo1[...]) + bo1[...]), Wo2[...]) + bo2[...]


def _node_mid_body(mn_ref, h_ref, hs_ref,
                   Wir, Whr, Wiz, Whz, Win, Whn, br, bz, bin_, bhn,
                   Ws1, bs1, Ws2, bs2, Wo1, bo1, Wo2, bo2,
                   hnew_ref, hso_ref, a_ref, b_ref):
    hn = _gru_core(mn_ref[...], h_ref[...],
                   Wir[...], Whr[...], Wiz[...], Whz[...], Win[...], Whn[...],
                   br[...], bz[...], bin_[...], bhn[...])
    hnew_ref[...] = hn
    hso_ref[...] = hs_ref[...] + hn
    a_ref[...] = _dot(jax.nn.relu(_dot(hn, Ws1[...]) + bs1[...]), Ws2[...]) + bs2[...]
    b_ref[...] = _dot(jax.nn.relu(_dot(hn, Wo1[...]) + bo1[...]), Wo2[...]) + bo2[...]


def _node_last_body(mn_ref, h_ref, hs_ref,
                    Wir, Whr, Wiz, Whz, Win, Whn, br, bz, bin_, bhn,
                    fg, fb, hfin_ref):
    hn = _gru_core(mn_ref[...], h_ref[...],
                   Wir[...], Whr[...], Wiz[...], Whz[...], Win[...], Whn[...],
                   br[...], bz[...], bin_[...], bhn[...])
    hfin_ref[...] = _ln(hs_ref[...] + hn, fg[...], fb[...])


def _edge_a_body(he_ref, W1, b1, W2, b2, mn_ref):
    x = he_ref[...]
    w = _dot(jax.nn.relu(_dot(x, W1[...]) + b1[...]), W2[...]) + b2[...]
    r = lax.broadcasted_iota(jnp.int32, (_SPB, _BLK), 0)
    c = lax.broadcasted_iota(jnp.int32, (_SPB, _BLK), 1)
    seg = ((c >= r * _SEG) & (c < (r + 1) * _SEG)).astype(_F32)
    mn_ref[...] = _dot(seg, w)


def _edge_b_body(sidx_ref, oidx_ref, a_ref, b_ref, he_ref, hs_ref,
                 g, b, Wir, Whr, Wiz, Whz, Win, Whn, br, bz, bin_, bhn,
                 *rest, final):
    if final:
        fg, fb, hef_ref = rest
    else:
        henew_ref, hso_ref = rest
    s = sidx_ref[0]
    o = oidx_ref[0]
    lanes = lax.broadcasted_iota(jnp.int32, (_BLK, _N), 1)
    ohs = (s == lanes).astype(_F32)
    oho = (o == lanes).astype(_F32)
    cc = _dot(ohs, a_ref[...]) + _dot(oho, b_ref[...])
    mp = _ln(cc, g[...], b[...])
    he = he_ref[...]
    hn = _gru_core(mp, he,
                   Wir[...], Whr[...], Wiz[...], Whz[...], Win[...], Whn[...],
                   br[...], bz[...], bin_[...], bhn[...])
    if final:
        hef_ref[...] = _ln(hs_ref[...] + hn, fg[...], fb[...])
    else:
        henew_ref[...] = hn
        hso_ref[...] = hs_ref[...] + hn


def _full(shape):
    nd = len(shape)
    return pl.BlockSpec(shape, lambda *_: (0,) * nd)


def _node_first(h, ws):
    out = [jax.ShapeDtypeStruct((_N, _ND), _F32)] * 2
    specs = [_full((_N, _ND))] + [_full(w.shape) for w in ws]
    return pl.pallas_call(
        _node_first_body,
        out_shape=out,
        in_specs=specs,
        out_specs=[_full((_N, _ND))] * 2,
    )(h, *ws)


def _node_mid(mn, h, hs, gw, ws):
    out = [jax.ShapeDtypeStruct((_N, _ND), _F32)] * 4
    specs = [_full((_N, _ND))] * 3 + [_full(w.shape) for w in gw + ws]
    return pl.pallas_call(
        _node_mid_body,
        out_shape=out,
        in_specs=specs,
        out_specs=[_full((_N, _ND))] * 4,
    )(mn, h, hs, *gw, *ws)


def _node_last(mn, h, hs, gw, fg, fb):
    specs = ([_full((_N, _ND))] * 3 + [_full(w.shape) for w in gw]
             + [_full((1, _ND))] * 2)
    return pl.pallas_call(
        _node_last_body,
        out_shape=jax.ShapeDtypeStruct((_N, _ND), _F32),
        in_specs=specs,
        out_specs=_full((_N, _ND)),
    )(mn, h, hs, *gw, fg, fb)


def _edge_a(he, W1, b1, W2, b2):
    return pl.pallas_call(
        _edge_a_body,
        grid=(_NB,),
        out_shape=jax.ShapeDtypeStruct((_N, _ND), _F32),
        in_specs=[
            pl.BlockSpec((_BLK, _ND), lambda i: (i, 0)),
            _full((_ND, _ND)), _full((1, _ND)), _full((_ND, _ND)), _full((1, _ND)),
        ],
        out_specs=pl.BlockSpec((_SPB, _ND), lambda i: (i, 0)),
    )(he, W1, b1, W2, b2)


def _edge_b(s3, o3, a, b, he, hs, lng, lnb, gw, final, fg=None, fb=None):
    body = functools.partial(_edge_b_body, final=final)
    in_specs = [
        pl.BlockSpec((1, _BLK, 1), lambda i: (i, 0, 0)),
        pl.BlockSpec((1, _BLK, 1), lambda i: (i, 0, 0)),
        _full((_N, _ND)), _full((_N, _ND)),
        pl.BlockSpec((_BLK, _ND), lambda i: (i, 0)),
        pl.BlockSpec((_BLK, _ND), lambda i: (i, 0)),
        _full((1, _ND)), _full((1, _ND)),
    ] + [_full(w.shape) for w in gw]
    args = [s3, o3, a, b, he, hs, lng, lnb, *gw]
    if final:
        in_specs += [_full((1, _ND)), _full((1, _ND))]
        args += [fg, fb]
        out_shape = jax.ShapeDtypeStruct((_E, _ND), _F32)
        out_specs = pl.BlockSpec((_BLK, _ND), lambda i: (i, 0))
    else:
        out_shape = [jax.ShapeDtypeStruct((_E, _ND), _F32)] * 2
        out_specs = [pl.BlockSpec((_BLK, _ND), lambda i: (i, 0))] * 2
    return pl.pallas_call(
        body,
        grid=(_NB,),
        out_shape=out_shape,
        in_specs=in_specs,
        out_specs=out_specs,
        compiler_params=pltpu.CompilerParams(
            vmem_limit_bytes=100 * 1024 * 1024),
    )(*args)


def _split_gru(Wih, Whh, bih, bhh):
    D = _ND
    Wir, Wiz, Win = (Wih[0:D].T, Wih[D:2 * D].T, Wih[2 * D:].T)
    Whr, Whz, Whn = (Whh[0:D].T, Whh[D:2 * D].T, Whh[2 * D:].T)
    br = (bih[0:D] + bhh[0:D]).reshape(1, D)
    bz = (bih[D:2 * D] + bhh[D:2 * D]).reshape(1, D)
    bin_ = bih[2 * D:].reshape(1, D)
    bhn = bhh[2 * D:].reshape(1, D)
    return [Wir, Whr, Wiz, Whz, Win, Whn, br, bz, bin_, bhn]


def kernel(h, h_edge, edge_index, phis_W1, phis_b1, phis_W2, phis_b2,
           phio_W1, phio_b1, phio_W2, phio_b2, phip_W1, phip_b1, phip_W2,
           phip_b2, ln_g, ln_b, gru_n_Wih, gru_n_Whh, gru_n_bih, gru_n_bhh,
           gru_e_Wih, gru_e_Whh, gru_e_bih, gru_e_bhh, final_ln_g, final_ln_b):
    s3 = edge_index[0].reshape(_NB, _BLK, 1)
    o3 = edge_index[1].reshape(_NB, _BLK, 1)
    fg = final_ln_g.reshape(1, _ND)
    fb = final_ln_b.reshape(1, _ND)

    hs = jnp.zeros((_N, _ND), _F32)
    hes = jnp.zeros((_E, _ND), _F32)
    mn = None
    he = h_edge
    hef = None

    for i in range(_NL):
        ws = [phis_W1[i], phis_b1[i].reshape(1, _ND), phis_W2[i],
              phis_b2[i].reshape(1, _ND), phio_W1[i], phio_b1[i].reshape(1, _ND),
              phio_W2[i], phio_b2[i].reshape(1, _ND)]
        ge = _split_gru(gru_e_Wih[i], gru_e_Whh[i], gru_e_bih[i], gru_e_bhh[i])

        if i == 0:
            a, b = _node_first(h, ws)
        else:
            gn = _split_gru(gru_n_Wih[i - 1], gru_n_Whh[i - 1],
                            gru_n_bih[i - 1], gru_n_bhh[i - 1])
            h, hs, a, b = _node_mid(mn, h, hs, gn, ws)

        mn = _edge_a(he, phip_W1[i], phip_b1[i].reshape(1, _ND),
                     phip_W2[i], phip_b2[i].reshape(1, _ND))

        lng = ln_g[i].reshape(1, _ND)
        lnb = ln_b[i].reshape(1, _ND)
        if i == _NL - 1:
            hef = _edge_b(s3, o3, a, b, he, hes, lng, lnb, ge,
                          final=True, fg=fg, fb=fb)
        else:
            he, hes = _edge_b(s3, o3, a, b, he, hes, lng, lnb, ge, final=False)

    gn = _split_gru(gru_n_Wih[_NL - 1], gru_n_Whh[_NL - 1],
                    gru_n_bih[_NL - 1], gru_n_bhh[_NL - 1])
    hf = _node_last(mn, h, hs, gn, fg, fb)
    return (hf, hef)
